# baseline (speedup 1.0000x reference)
import jax
import jax.numpy as jnp
from jax.experimental import pallas as pl
from jax.experimental.pallas import tpu as pltpu

H = 16
D = 64
T = 128
NT = 16
S = T * NT
HD = H * D
SCALE = D ** -0.5
QB = 256
CB = 512
NCB = S // CB


def _mm_kernel(a_ref, b_ref, o_ref):
    o_ref[...] = jnp.dot(a_ref[...], b_ref[...],
                         preferred_element_type=jnp.float32).astype(o_ref.dtype)


def _mm(a, b, bn, out_dtype):
    m, k = a.shape
    _, n = b.shape
    return pl.pallas_call(
        _mm_kernel,
        grid=(n // bn,),
        in_specs=[pl.BlockSpec((m, k), lambda t: (0, 0)),
                  pl.BlockSpec((k, bn), lambda t: (0, t))],
        out_specs=pl.BlockSpec((m, bn), lambda t: (0, t)),
        out_shape=jax.ShapeDtypeStruct((m, n), out_dtype),
    )(a, b)


def _select_kernel(bm_ref, am_ref):
    bm = bm_ref[...]
    r = jax.lax.broadcasted_iota(jnp.int32, (H * NT, NT), 0)
    i_row = jax.lax.rem(r, NT)
    j = jax.lax.broadcasted_iota(jnp.int32, (H * NT, NT), 1)
    active = ((j < i_row) & (bm != 0)) | (j == i_row)
    add = jnp.where(active, 0.0, -1e30).astype(jnp.float32)
    g = (jax.lax.broadcasted_iota(jnp.int32, (NT, S), 0)
         == jax.lax.broadcasted_iota(jnp.int32, (NT, S), 1) // T)
    am_ref[...] = jnp.dot(add, g.astype(jnp.float32),
                          preferred_element_type=jnp.float32)


def _select(bm2):
    return pl.pallas_call(
        _select_kernel,
        out_shape=jax.ShapeDtypeStruct((H * NT, S), jnp.float32),
    )(bm2)


def _attn_kernel(q_ref, k_ref, v_ref, am_ref, o_ref, acc_ref, m_ref, l_ref):
    i = pl.program_id(1)
    c = pl.program_id(2)

    @pl.when(c == 0)
    def _():
        acc_ref[...] = jnp.zeros_like(acc_ref)
        m_ref[...] = jnp.full_like(m_ref, -1e37)
        l_ref[...] = jnp.zeros_like(l_ref)

    @pl.when(2 * c <= i)
    def _():
        q = q_ref[0]
        k = k_ref[0]
        s = jax.lax.dot_general(q, k, (((1,), (1,)), ((), ())),
                                preferred_element_type=jnp.float32)
        am = am_ref[:, 0, :]
        amx = jnp.concatenate(
            [jnp.broadcast_to(am[t:t + 1], (T, CB)) for t in range(QB // T)], 0)
        grow = i * QB + jax.lax.broadcasted_iota(jnp.int32, (QB, CB), 0)
        gcol = c * CB + jax.lax.broadcasted_iota(jnp.int32, (QB, CB), 1)
        s = jnp.where(gcol <= grow, s + amx, -1e30)
        m_prev = m_ref[...]
        m_new = jnp.maximum(m_prev, jnp.max(s, axis=1, keepdims=True))
        alpha = jnp.exp(m_prev - m_new)
        p = jnp.exp(s - m_new)
        l_ref[...] = l_ref[...] * alpha + jnp.sum(p, axis=1, keepdims=True)
        acc_ref[...] = acc_ref[...] * alpha + jax.lax.dot_general(
            p.astype(jnp.bfloat16), v_ref[0], (((1,), (0,)), ((), ())),
            preferred_element_type=jnp.float32)
        m_ref[...] = m_new

    @pl.when(c == NCB - 1)
    def _():
        o_ref[0] = (acc_ref[...] / l_ref[...]).astype(o_ref.dtype)


def _attend(am, qkv):
    nq = NT * T // QB

    def cmap(base):
        def f(h, i, c):
            return (base + h, jnp.minimum(c, i // 2), 0)
        return f

    def ammap(h, i, c):
        return (h * nq + i, 0, jnp.minimum(c, i // 2))

    return pl.pallas_call(
        _attn_kernel,
        grid=(H, nq, NCB),
        in_specs=[pl.BlockSpec((1, QB, D), lambda h, i, c: (h, i, 0)),
                  pl.BlockSpec((1, CB, D), cmap(H)),
                  pl.BlockSpec((1, CB, D), cmap(2 * H)),
                  pl.BlockSpec((QB // T, 1, CB), ammap)],
        out_specs=pl.BlockSpec((1, QB, D), lambda h, i, c: (h, i, 0)),
        out_shape=jax.ShapeDtypeStruct((H, S, D), jnp.bfloat16),
        scratch_shapes=[pltpu.VMEM((QB, D), jnp.float32),
                        pltpu.VMEM((QB, 1), jnp.float32),
                        pltpu.VMEM((QB, 1), jnp.float32)],
        compiler_params=pltpu.CompilerParams(
            dimension_semantics=("parallel", "parallel", "arbitrary")),
    )(qkv, qkv, qkv, am)


def _oproj_kernel(a_ref, b_ref, o_ref):
    @pl.when(pl.program_id(0) == 0)
    def _():
        o_ref[...] = jnp.zeros_like(o_ref)

    o_ref[...] += jnp.dot(a_ref[0], b_ref[0],
                          preferred_element_type=jnp.float32)


def _oproj(attn, wo3):
    e = wo3.shape[2]
    return pl.pallas_call(
        _oproj_kernel,
        grid=(H,),
        in_specs=[pl.BlockSpec((1, S, D), lambda h: (h, 0, 0)),
                  pl.BlockSpec((1, D, e), lambda h: (h, 0, 0))],
        out_specs=pl.BlockSpec((S, e), lambda h: (0, 0)),
        out_shape=jax.ShapeDtypeStruct((S, e), jnp.float32),
    )(attn, wo3)


def kernel(x, block_mask, Wq, Wk, Wv, Wo):
    batch, _, e = x.shape
    xb = x.reshape(S, e).astype(jnp.bfloat16)
    w = jnp.concatenate([Wq * SCALE, Wk, Wv], axis=1).astype(jnp.bfloat16)
    qkv = _mm(xb, w, 512, jnp.bfloat16)
    qkvt = qkv.reshape(S, 3 * H, D).transpose(1, 0, 2)
    bm2 = block_mask.reshape(H * NT, NT).astype(jnp.int32)
    am = _select(bm2)
    attn = _attend(am.reshape(H * NT, 1, S), qkvt)
    out = _oproj(attn, Wo.reshape(H, D, -1).astype(jnp.bfloat16))
    return out.reshape(batch, S, -1)


# speedup vs baseline: 1.8536x; 1.8536x over previous
import jax
import jax.numpy as jnp
from jax.experimental import pallas as pl
from jax.experimental.pallas import tpu as pltpu

H = 16
D = 64
T = 128
NT = 16
S = T * NT
HD = H * D
SCALE = D ** -0.5
QB = 256


def _mm_kernel(a_ref, b_ref, o_ref):
    o_ref[...] = jnp.dot(a_ref[...], b_ref[...],
                         preferred_element_type=jnp.float32).astype(o_ref.dtype)


def _mm(a, b, bn, out_dtype):
    m, k = a.shape
    _, n = b.shape
    return pl.pallas_call(
        _mm_kernel,
        grid=(n // bn,),
        in_specs=[pl.BlockSpec((m, k), lambda t: (0, 0)),
                  pl.BlockSpec((k, bn), lambda t: (0, t))],
        out_specs=pl.BlockSpec((m, bn), lambda t: (0, t)),
        out_shape=jax.ShapeDtypeStruct((m, n), out_dtype),
    )(a, b)


def _select_kernel(bm_ref, am_ref):
    bm = bm_ref[...]
    r = jax.lax.broadcasted_iota(jnp.int32, (H * NT, NT), 0)
    i_row = jax.lax.rem(r, NT)
    j = jax.lax.broadcasted_iota(jnp.int32, (H * NT, NT), 1)
    active = ((j < i_row) & (bm != 0)) | (j == i_row)
    add = jnp.where(active, 0.0, -1e30).astype(jnp.float32)
    g = (jax.lax.broadcasted_iota(jnp.int32, (NT, S), 0)
         == jax.lax.broadcasted_iota(jnp.int32, (NT, S), 1) // T)
    am_ref[...] = jnp.dot(add, g.astype(jnp.float32),
                          preferred_element_type=jnp.float32)


def _select(bm2):
    return pl.pallas_call(
        _select_kernel,
        out_shape=jax.ShapeDtypeStruct((H * NT, S), jnp.float32),
    )(bm2)


def _attn_body(i0, ext):
    def body(q_ref, k_ref, v_ref, am_ref, o_ref):
        i = pl.program_id(1) + i0
        q = q_ref[0]
        k = k_ref[0]
        s = jax.lax.dot_general(q, k, (((1,), (1,)), ((), ())),
                                preferred_element_type=jnp.float32)
        am = am_ref[:, 0, :]
        amx = jnp.concatenate(
            [jnp.broadcast_to(am[t:t + 1], (T, ext)) for t in range(QB // T)],
            0)
        grow = i * QB + jax.lax.broadcasted_iota(jnp.int32, (QB, ext), 0)
        gcol = jax.lax.broadcasted_iota(jnp.int32, (QB, ext), 1)
        s = jnp.where(gcol <= grow, s + amx, -1e30)
        m = jnp.max(s, axis=1, keepdims=True)
        p = jnp.exp(s - m)
        l = jnp.sum(p, axis=1, keepdims=True)
        o = jnp.dot(p.astype(jnp.bfloat16), v_ref[0],
                    preferred_element_type=jnp.float32) / l
        o_ref[0] = o.astype(o_ref.dtype)
    return body


def _attend_part(am, qkv, i0, nqc, ext):
    nq = NT * T // QB
    return pl.pallas_call(
        _attn_body(i0, ext),
        grid=(H, nqc),
        in_specs=[pl.BlockSpec((1, QB, D), lambda h, i: (h, i0 + i, 0)),
                  pl.BlockSpec((1, ext, D), lambda h, i: (H + h, 0, 0)),
                  pl.BlockSpec((1, ext, D), lambda h, i: (2 * H + h, 0, 0)),
                  pl.BlockSpec((QB // T, 1, ext),
                               lambda h, i: (h * nq + i0 + i, 0, 0))],
        out_specs=pl.BlockSpec((1, QB, D), lambda h, i: (h, i, 0)),
        out_shape=jax.ShapeDtypeStruct((H, nqc * QB, D), jnp.bfloat16),
    )(qkv, qkv, qkv, am)


def _attend(am, qkv):
    parts = [_attend_part(am, qkv, i0, 2, (i0 + 2) * QB) for i0 in (0, 2, 4, 6)]
    return jnp.concatenate(parts, axis=1)


def _oproj_kernel(a_ref, b_ref, o_ref):
    @pl.when(pl.program_id(0) == 0)
    def _():
        o_ref[...] = jnp.zeros_like(o_ref)

    o_ref[...] += jnp.dot(a_ref[0], b_ref[0],
                          preferred_element_type=jnp.float32)


def _oproj(attn, wo3):
    e = wo3.shape[2]
    return pl.pallas_call(
        _oproj_kernel,
        grid=(H,),
        in_specs=[pl.BlockSpec((1, S, D), lambda h: (h, 0, 0)),
                  pl.BlockSpec((1, D, e), lambda h: (h, 0, 0))],
        out_specs=pl.BlockSpec((S, e), lambda h: (0, 0)),
        out_shape=jax.ShapeDtypeStruct((S, e), jnp.float32),
    )(attn, wo3)


def kernel(x, block_mask, Wq, Wk, Wv, Wo):
    batch, _, e = x.shape
    xb = x.reshape(S, e).astype(jnp.bfloat16)
    w = jnp.concatenate([Wq * SCALE, Wk, Wv], axis=1).astype(jnp.bfloat16)
    qkv = _mm(xb, w, 512, jnp.bfloat16)
    qkvt = qkv.reshape(S, 3 * H, D).transpose(1, 0, 2)
    bm2 = block_mask.reshape(H * NT, NT).astype(jnp.int32)
    am = _select(bm2)
    attn = _attend(am.reshape(H * NT, 1, S), qkvt)
    out = _oproj(attn, Wo.reshape(H, D, -1).astype(jnp.bfloat16))
    return out.reshape(batch, S, -1)
